# baseline (speedup 1.0000x reference)
import jax
import jax.numpy as jnp
from jax import lax
from jax.experimental import pallas as pl
from jax.experimental.pallas import tpu as pltpu
from jax.experimental.pallas import tpu_sc as plsc

NC = 2
NS = 16
NW = NC * NS
BB = 128
NBUF = 4
DP = 128
L = 16


def _make_gather(nb: int, nt: int, d: int, interpret: bool = False):
    assert nb == NW * BB and nt % NBUF == 0
    mesh = plsc.VectorSubcoreMesh(
        core_axis_name="c", subcore_axis_name="s", num_cores=NC, num_subcores=NS
    )

    def body(idx_hbm, table_hbm, out_hbm, idx_v, *bufs):
        rows = bufs[:NBUF]
        stage = bufs[NBUF:2 * NBUF]
        gsem = bufs[2 * NBUF:3 * NBUF]
        osem = bufs[3 * NBUF:]
        wid = lax.axis_index("s") * NC + lax.axis_index("c")
        col0 = wid * BB
        pltpu.sync_copy(idx_hbm.at[:, pl.ds(col0, BB)], idx_v)
        iota = lax.iota(jnp.int32, L)

        def fire_gather(t, b):
            pltpu.async_copy(table_hbm.at[idx_v.at[t]], rows[b], gsem[b])

        def transpose_into_stage(b):
            @pl.loop(0, d // 4)
            def _(cq):
                for ci in range(4):
                    c = cq * 4 + ci
                    cvec = jnp.full((L,), c, jnp.int32)
                    for ib in range(BB // L):
                        v = plsc.load_gather(rows[b], [ib * L + iota, cvec])
                        stage[b][c, pl.ds(ib * L, L)] = v

        for t in range(NBUF - 1):
            fire_gather(t, t)

        @pl.loop(0, nt // NBUF)
        def _(tl):
            for b in range(NBUF):
                t = tl * NBUF + b
                pltpu.make_async_copy(
                    table_hbm.at[pl.ds(0, BB)], rows[b], gsem[b]
                ).wait()
                @pl.when(t >= NBUF)
                def _():
                    pltpu.make_async_copy(
                        table_hbm.at[pl.ds(0, d)], stage[b], osem[b]
                    ).wait()

                transpose_into_stage(b)
                pltpu.async_copy(
                    stage[b], out_hbm.at[t].at[:, pl.ds(col0, BB)], osem[b]
                )

                @pl.when(t + NBUF - 1 < nt)
                def _():
                    fire_gather(t + NBUF - 1, (b + NBUF - 1) % NBUF)

        for b in range(NBUF):
            pltpu.make_async_copy(
                table_hbm.at[pl.ds(0, d)], stage[b], osem[b]
            ).wait()

    return pl.kernel(
        body,
        out_type=jax.ShapeDtypeStruct((nt, d, nb), jnp.float32),
        mesh=mesh,
        scratch_types=(
            (pltpu.VMEM((nt, BB), jnp.int32),)
            + tuple(pltpu.VMEM((BB, DP), jnp.float32) for _ in range(NBUF))
            + tuple(pltpu.VMEM((d, BB), jnp.float32) for _ in range(NBUF))
            + tuple(pltpu.SemaphoreType.DMA for _ in range(2 * NBUF))
        ),
        compiler_params=pltpu.CompilerParams(
            use_tc_tiling_on_sc=True, needs_layout_passes=False
        ),
        interpret=interpret,
    )


def kernel(token_ids, embedding_matrix):
    nb, nt = token_ids.shape
    d = embedding_matrix.shape[1]
    ids_t = token_ids.astype(jnp.int32).T
    table_p = jnp.pad(embedding_matrix, ((0, 0), (0, DP - d)))
    out_t = _make_gather(nb, nt, d)(ids_t, table_p)
    return jnp.transpose(out_t, (2, 0, 1))


# speedup vs baseline: 1.7463x; 1.7463x over previous
import jax
import jax.numpy as jnp
from jax import lax
from jax.experimental import pallas as pl
from jax.experimental.pallas import tpu as pltpu
from jax.experimental.pallas import tpu_sc as plsc

NC = 2
NS = 16
NW = NC * NS
GRP = 128
NBUF = 5
DP = 128


def _make_gather(n_ids: int, d: int, out_shape, interpret: bool = False):
    assert n_ids % (NW * GRP * NBUF) == 0
    ng = n_ids // (NW * GRP)
    b_per_w = ng * GRP
    mesh = plsc.VectorSubcoreMesh(
        core_axis_name="c", subcore_axis_name="s", num_cores=NC, num_subcores=NS
    )

    def body(idx_hbm, table_hbm, out_hbm, idx_v, *bufs):
        rows = bufs[:NBUF]
        gsem = bufs[NBUF:2 * NBUF]
        osem = bufs[2 * NBUF:]
        out_flat = out_hbm.reshape(n_ids, DP)
        wid = lax.axis_index("s") * NC + lax.axis_index("c")
        wbase = wid * b_per_w
        pltpu.sync_copy(idx_hbm.at[pl.ds(wid * ng, ng)], idx_v)

        def fire_gather(g, b):
            pltpu.async_copy(table_hbm.at[idx_v.at[g]], rows[b], gsem[b])

        for g in range(NBUF - 1):
            fire_gather(g, g)

        @pl.loop(0, ng // NBUF)
        def _(gl):
            for b in range(NBUF):
                g = gl * NBUF + b
                pltpu.make_async_copy(
                    out_flat.at[pl.ds(0, GRP)], rows[b], gsem[b]
                ).wait()
                pltpu.async_copy(
                    rows[b], out_flat.at[pl.ds(wbase + g * GRP, GRP)], osem[b]
                )
                bf = (b + NBUF - 1) % NBUF

                @pl.when(g >= 1)
                def _():
                    pltpu.make_async_copy(
                        out_flat.at[pl.ds(0, GRP)], rows[bf], osem[bf]
                    ).wait()

                @pl.when(g + NBUF - 1 < ng)
                def _():
                    fire_gather(g + NBUF - 1, bf)

        lb = (ng - 1) % NBUF
        pltpu.make_async_copy(
            out_flat.at[pl.ds(0, GRP)], rows[lb], osem[lb]
        ).wait()

    return pl.kernel(
        body,
        out_type=jax.ShapeDtypeStruct(out_shape, jnp.float32),
        mesh=mesh,
        scratch_types=(
            (pltpu.VMEM((ng, GRP), jnp.int32),)
            + tuple(pltpu.VMEM((GRP, DP), jnp.float32) for _ in range(NBUF))
            + tuple(pltpu.SemaphoreType.DMA for _ in range(2 * NBUF))
        ),
        compiler_params=pltpu.CompilerParams(use_tc_tiling_on_sc=True),
        interpret=interpret,
    )


def kernel(token_ids, embedding_matrix):
    nb, nt = token_ids.shape
    n = nb * nt
    d = embedding_matrix.shape[1]
    idx = token_ids.astype(jnp.int32).reshape(n // GRP, GRP)
    table_p = jnp.pad(embedding_matrix, ((0, 0), (0, DP - d)))
    out_p = _make_gather(n, d, (nb, nt, DP))(idx, table_p)
    return out_p[:, :, :d]
